# Initial kernel scaffold; baseline (speedup 1.0000x reference)
import jax
import jax.numpy as jnp
from jax.experimental import pallas as pl


def kernel(objs, triples, obj_to_img, obj_emb, pred_emb, gconv_params, att_w, att_b):
    raise NotImplementedError("write your pallas kernel here")


# speedup vs baseline: 1.0792x; 1.0792x over previous
import functools

import jax
import jax.numpy as jnp
from jax import lax
from jax.experimental import pallas as pl
from jax.experimental.pallas import tpu as pltpu

O = 10000
T = 160000
N_IMG = 64
D = 128
H = 512
O_PAD = 10240
TB = 640
OB = 1280
NEG = -1e30


def _edge_mlp_body(sv_ref, pv_ref, ov_ref, was_ref, wap_ref, wao_ref, b1a_ref,
                   w1b_ref, b1b_ref, m_ref, np_ref):
    h = (jnp.dot(sv_ref[...], was_ref[...], preferred_element_type=jnp.float32)
         + jnp.dot(pv_ref[...], wap_ref[...], preferred_element_type=jnp.float32)
         + jnp.dot(ov_ref[...], wao_ref[...], preferred_element_type=jnp.float32)
         + b1a_ref[0:1, :])
    h = jnp.maximum(h, 0.0)
    nt = jnp.dot(h, w1b_ref[...], preferred_element_type=jnp.float32) + b1b_ref[0:1, :]
    nt = jnp.maximum(nt, 0.0)
    m_ref[0] = nt[:, :H]
    np_ref[...] = nt[:, H:H + D]
    m_ref[1] = nt[:, H + D:]


def _edge_mlp(sv, pv, ov, w1a, b1a, w1b, b1b):
    was, wap, wao = w1a[:D], w1a[D:2 * D], w1a[2 * D:]
    grid = (T // TB,)
    full = lambda i: (0, 0)
    m, new_p = pl.pallas_call(
        _edge_mlp_body,
        grid=grid,
        in_specs=[
            pl.BlockSpec((TB, D), lambda i: (i, 0)),
            pl.BlockSpec((TB, D), lambda i: (i, 0)),
            pl.BlockSpec((TB, D), lambda i: (i, 0)),
            pl.BlockSpec((D, H), full),
            pl.BlockSpec((D, H), full),
            pl.BlockSpec((D, H), full),
            pl.BlockSpec((1, H), full),
            pl.BlockSpec((H, 2 * H + D), full),
            pl.BlockSpec((1, 2 * H + D), full),
        ],
        out_specs=[
            pl.BlockSpec((2, TB, H), lambda i: (0, i, 0)),
            pl.BlockSpec((TB, D), lambda i: (i, 0)),
        ],
        out_shape=[
            jax.ShapeDtypeStruct((2, T, H), jnp.float32),
            jax.ShapeDtypeStruct((T, D), jnp.float32),
        ],
    )(sv, pv, ov, was, wap, wao, b1a.reshape(1, H), w1b,
      b1b.reshape(1, 2 * H + D))
    return m, new_p


def _node_mlp_body(pooled_ref, invc_ref, w2a_ref, b2a_ref, w2b_ref, b2b_ref,
                   out_ref):
    x = pooled_ref[...] * invc_ref[...]
    h2 = jnp.dot(x, w2a_ref[...], preferred_element_type=jnp.float32) + b2a_ref[0:1, :]
    h2 = jnp.maximum(h2, 0.0)
    y = jnp.dot(h2, w2b_ref[...], preferred_element_type=jnp.float32) + b2b_ref[0:1, :]
    out_ref[...] = jnp.maximum(y, 0.0)


def _node_mlp(pooled, invc_full, w2a, b2a, w2b, b2b):
    grid = (O_PAD // OB,)
    full = lambda i: (0, 0)
    return pl.pallas_call(
        _node_mlp_body,
        grid=grid,
        in_specs=[
            pl.BlockSpec((OB, H), lambda i: (i, 0)),
            pl.BlockSpec((OB, H), lambda i: (i, 0)),
            pl.BlockSpec((H, H), full),
            pl.BlockSpec((1, H), full),
            pl.BlockSpec((H, D), full),
            pl.BlockSpec((1, D), full),
        ],
        out_specs=pl.BlockSpec((OB, D), lambda i: (i, 0)),
        out_shape=jax.ShapeDtypeStruct((O_PAD, D), jnp.float32),
    )(pooled, invc_full, w2a, b2a.reshape(1, H), w2b, b2b.reshape(1, D))


def _att_body(ov_ref, oimc_ref, oimr_ref, attw_ref, attb_ref, out_ref):
    ov = ov_ref[...]
    sc8 = jnp.dot(ov, attw_ref[...], preferred_element_type=jnp.float32)
    scores = jnp.tanh(sc8[:, 0:1] + attb_ref[0, 0])
    oimc = oimc_ref[...][:, 0:1]
    oh1b = lax.broadcasted_iota(jnp.int32, (O, N_IMG), 1) == oimc
    oh1f = oh1b.astype(jnp.float32)
    sb = jnp.broadcast_to(scores, (O, N_IMG))
    sm = jnp.max(jnp.where(oh1b, sb, NEG), axis=0, keepdims=True)
    smax_obj = lax.dot_general(oh1f, sm, (((1,), (1,)), ((), ())),
                               preferred_element_type=jnp.float32)
    e = jnp.exp(scores - smax_obj)
    eb = jnp.broadcast_to(e, (O, N_IMG))
    den = jnp.sum(jnp.where(oh1b, eb, 0.0), axis=0, keepdims=True)
    den_obj = lax.dot_general(oh1f, den, (((1,), (1,)), ((), ())),
                              preferred_element_type=jnp.float32)
    alpha = e / den_obj
    w = ov * alpha
    oh2f = (lax.broadcasted_iota(jnp.int32, (N_IMG, O), 0)
            == oimr_ref[0:1, :]).astype(jnp.float32)
    pooled = jnp.dot(oh2f, w, preferred_element_type=jnp.float32)
    gv = jnp.dot(oh1f, pooled, preferred_element_type=jnp.float32)
    out_ref[...] = jnp.concatenate([ov, gv], axis=1)


def _attention(ov_pad, obj_to_img, att_w, att_b):
    oimc = jnp.broadcast_to(obj_to_img.reshape(O, 1), (O, 8))
    oimr = jnp.broadcast_to(obj_to_img.reshape(1, O), (8, O))
    attw8 = jnp.broadcast_to(att_w, (D, 8))
    attb = jnp.broadcast_to(att_b.reshape(1, 1), (8, 128))
    return pl.pallas_call(
        _att_body,
        in_specs=[
            pl.BlockSpec((O, D), lambda: (0, 0)),
            pl.BlockSpec((O, 8), lambda: (0, 0)),
            pl.BlockSpec((8, O), lambda: (0, 0)),
            pl.BlockSpec((D, 8), lambda: (0, 0)),
            pl.BlockSpec((8, 128), lambda: (0, 0)),
        ],
        out_specs=pl.BlockSpec((O, 2 * D), lambda: (0, 0)),
        out_shape=jax.ShapeDtypeStruct((O, 2 * D), jnp.float32),
    )(ov_pad[:O], oimc, oimr, attw8, attb)


def kernel(objs, triples, obj_to_img, obj_emb, pred_emb, gconv_params,
           att_w, att_b):
    s = triples[:, 0]
    p = triples[:, 1]
    o = triples[:, 2]
    objs_pad = jnp.concatenate([objs, jnp.zeros((O_PAD - O,), objs.dtype)])

    dest = jnp.concatenate([s, o])
    counts = jnp.bincount(dest, length=O_PAD).astype(jnp.float32)
    invc = 1.0 / jnp.maximum(counts, 1.0)
    invc_full = jnp.broadcast_to(invc.reshape(O_PAD, 1), (O_PAD, H))

    ovp = obj_emb[objs_pad]
    pv = pred_emb[p]

    for lp in gconv_params:
        w1a, b1a, w1b, b1b, w2a, b2a, w2b, b2b = lp
        sv = ovp[s]
        ovv = ovp[o]
        m, pv = _edge_mlp(sv, pv, ovv, w1a, b1a, w1b, b1b)
        pooled = (jnp.zeros((O_PAD, H), jnp.float32)
                  .at[s].add(m[0]).at[o].add(m[1]))
        ovp = _node_mlp(pooled, invc_full, w2a, b2a, w2b, b2b)

    return _attention(ovp, obj_to_img, att_w, att_b)
